# Initial kernel scaffold; baseline (speedup 1.0000x reference)
import jax
import jax.numpy as jnp
from jax.experimental import pallas as pl


def kernel(x, coords, W):
    raise NotImplementedError("write your pallas kernel here")


# speedup vs baseline: 20.2526x; 20.2526x over previous
import functools

import jax
import jax.numpy as jnp
from jax import lax
from jax.experimental import pallas as pl
from jax.experimental.pallas import tpu as pltpu
from jax.experimental.pallas import tpu_sc as plsc

B = 8
C = 64
N = 8192
S = 1024
K = 16


_FCH = 128


def _fps_body(ct_ref, sel_ref):
    X = ct_ref[0]
    Y = ct_ref[1]
    Z = ct_ref[2]
    lane = lax.broadcasted_iota(jnp.int32, (B, N), 1)
    lane128 = lax.broadcasted_iota(jnp.int32, (B, _FCH), 1)

    lx = X[:, 0:1]
    ly = Y[:, 0:1]
    lz = Z[:, 0:1]
    zero_ch = jnp.zeros((B, _FCH), dtype=jnp.float32)
    put0 = lane128 == 0
    ax = jnp.where(put0, lx, zero_ch)
    ay = jnp.where(put0, ly, zero_ch)
    az = jnp.where(put0, lz, zero_ch)
    dists = jnp.full((B, N), 1e10, dtype=jnp.float32)

    def body(j, carry):
        dists, lx, ly, lz, ax, ay, az = carry
        dx = X - lx
        dy = Y - ly
        dz = Z - lz
        d = dx * dx + dy * dy + dz * dz
        dists = jnp.minimum(dists, d)
        m = jnp.max(dists, axis=1, keepdims=True)
        cand = jnp.where(dists == m, lane, N)
        nxt = jnp.min(cand, axis=1, keepdims=True)
        one = lane == nxt
        nlx = jnp.sum(jnp.where(one, X, 0.0), axis=1, keepdims=True)
        nly = jnp.sum(jnp.where(one, Y, 0.0), axis=1, keepdims=True)
        nlz = jnp.sum(jnp.where(one, Z, 0.0), axis=1, keepdims=True)
        put = lane128 == j
        ax = jnp.where(put, nlx, ax)
        ay = jnp.where(put, nly, ay)
        az = jnp.where(put, nlz, az)
        return (dists, nlx, nly, nlz, ax, ay, az)

    for ch in range(S // _FCH):
        start = 1 if ch == 0 else 0
        carry = (dists, lx, ly, lz, ax, ay, az)
        carry = lax.fori_loop(start, _FCH, body, carry)
        dists, lx, ly, lz, ax, ay, az = carry
        cs = ch * _FCH
        sel_ref[0, :, cs:cs + _FCH] = ax
        sel_ref[1, :, cs:cs + _FCH] = ay
        sel_ref[2, :, cs:cs + _FCH] = az
        ax = zero_ch
        ay = zero_ch
        az = zero_ch


def _fps(ct):
    return pl.pallas_call(
        _fps_body,
        out_shape=jax.ShapeDtypeStruct((3, B, S), jnp.float32),
    )(ct)


_ZT = 2048


def _z_body(x_ref, w_ref, z_ref):
    xb = x_ref[0]
    z_ref[0] = lax.dot_general(
        xb, w_ref[...], (((0,), (1,)), ((), ())),
        preferred_element_type=jnp.float32)


def _zrows(x, W):
    return pl.pallas_call(
        _z_body,
        grid=(B, N // _ZT),
        in_specs=[
            pl.BlockSpec((1, C, _ZT), lambda b, n: (b, 0, n)),
            pl.BlockSpec((C, C), lambda b, n: (0, 0)),
        ],
        out_specs=pl.BlockSpec((1, _ZT, C), lambda b, n: (b, n, 0)),
        out_shape=jax.ShapeDtypeStruct((B, N, C), jnp.float32),
    )(x, W)


_SQ = 128


def _knn_body(q_ref, k_ref, idx_ref, d2_s):
    b = pl.program_id(0)
    q = q_ref[0]
    keys = k_ref[0]
    g = lax.dot_general(q, keys, (((1,), (0,)), ((), ())),
                        preferred_element_type=jnp.float32)
    qq = jnp.sum(q * q, axis=1, keepdims=True)
    pp = jnp.sum(keys * keys, axis=0, keepdims=True)
    d2_s[...] = qq + pp - 2.0 * g
    lane = lax.broadcasted_iota(jnp.int32, (_SQ, N), 1)
    base = b * N
    for j in range(K):
        d2 = d2_s[...]
        m = jnp.min(d2, axis=1, keepdims=True)
        cand = jnp.where(d2 == m, lane, N)
        nxt = jnp.min(cand, axis=1, keepdims=True)
        idx_ref[0, :, j:j + 1] = nxt + base
        d2_s[...] = jnp.where(lane == nxt, jnp.float32(jnp.inf), d2)


def _knn(fpsT, coords):
    return pl.pallas_call(
        _knn_body,
        grid=(B, S // _SQ),
        in_specs=[
            pl.BlockSpec((1, _SQ, 3), lambda b, s: (b, s, 0)),
            pl.BlockSpec((1, 3, N), lambda b, s: (b, 0, 0)),
        ],
        out_specs=pl.BlockSpec((1, _SQ, K), lambda b, s: (b, s, 0)),
        out_shape=jax.ShapeDtypeStruct((B, S, K), jnp.int32),
        scratch_shapes=[pltpu.VMEM((_SQ, N), jnp.float32)],
    )(fpsT, coords)


_NW = 32
_RPW = (B * S) // _NW
_GRP = 8
_NG = _RPW // _GRP


def _gmax_body(z_hbm, idx_hbm, y_hbm, idx_v, rows_v, out_v, sem):
    cid = lax.axis_index("c")
    sid = lax.axis_index("s")
    wid = sid * 2 + cid
    base = wid * _RPW
    pltpu.sync_copy(idx_hbm.at[pl.ds(base * K, _RPW * K)], idx_v)

    def group(g, _):
        ib = pl.multiple_of(g * (_GRP * K), 8)
        cp = pltpu.async_copy(
            z_hbm.at[idx_v.at[pl.ds(ib, _GRP * K)]], rows_v, sem)
        cp.wait()
        ob = g * _GRP
        for r in range(_GRP):
            for c4 in range(C // 16):
                cs = c4 * 16
                acc = rows_v[r * K, pl.ds(cs, 16)]
                for t in range(1, K):
                    acc = jnp.maximum(acc, rows_v[r * K + t, pl.ds(cs, 16)])
                out_v[ob + r, pl.ds(cs, 16)] = acc
        return 0

    lax.fori_loop(0, _NG, group, 0)
    pltpu.sync_copy(out_v, y_hbm.at[pl.ds(base, _RPW)])


@functools.cache
def _gmax_kernel():
    return pl.kernel(
        _gmax_body,
        out_type=jax.ShapeDtypeStruct((B * S, C), jnp.float32),
        mesh=plsc.VectorSubcoreMesh(
            core_axis_name="c", subcore_axis_name="s"),
        scratch_types=[
            pltpu.VMEM((_RPW * K,), jnp.int32),
            pltpu.VMEM((_GRP * K, C), jnp.float32),
            pltpu.VMEM((_RPW, C), jnp.float32),
            pltpu.SemaphoreType.DMA,
        ],
        compiler_params=pltpu.CompilerParams(use_tc_tiling_on_sc=False),
    )


@jax.jit
def kernel(x, coords, W):
    ct = jnp.transpose(coords, (1, 0, 2))
    fps_sel = _fps(ct)
    fps_coords = jnp.transpose(fps_sel, (1, 0, 2))
    fpsT = jnp.transpose(fps_sel, (1, 2, 0))
    z = _zrows(x, W)
    knn = _knn(fpsT, coords)
    y_rows = _gmax_kernel()(z.reshape(B * N, C), knn.reshape(B * S * K))
    y = jnp.transpose(y_rows.reshape(B, S, C), (0, 2, 1))
    return (y, fps_coords)


# speedup vs baseline: 32.8263x; 1.6208x over previous
import functools

import jax
import jax.numpy as jnp
from jax import lax
from jax.experimental import pallas as pl
from jax.experimental.pallas import tpu as pltpu
from jax.experimental.pallas import tpu_sc as plsc

B = 8
C = 64
N = 8192
S = 1024
K = 16

_NCH = N // 128
_FCH = 128
_BIGI = 0x7FFFFFFF


def _fps_body(ct_ref, x_ref, w_ref, sel_ref, z_ref, d_ref):
    for b in range(B):
        z_ref[b] = lax.dot_general(
            x_ref[b], w_ref[...], (((0,), (1,)), ((), ())),
            preferred_element_type=jnp.float32)

    lane128 = lax.broadcasted_iota(jnp.int32, (B, 128), 1)
    d_ref[...] = jnp.full((B, N), 1e10, dtype=jnp.float32)

    lx = ct_ref[0, :, 0:1]
    ly = ct_ref[1, :, 0:1]
    lz = ct_ref[2, :, 0:1]
    zero_ch = jnp.zeros((B, _FCH), dtype=jnp.float32)
    put0 = lane128 == 0
    ax = jnp.where(put0, lx, zero_ch)
    ay = jnp.where(put0, ly, zero_ch)
    az = jnp.where(put0, lz, zero_ch)

    def body(j, carry):
        lx, ly, lz, ax, ay, az = carry
        macc = jnp.full((B, 128), -jnp.inf, dtype=jnp.float32)
        bx = zero_ch
        by = zero_ch
        bz = zero_ch
        bc = jnp.zeros((B, 128), dtype=jnp.int32)
        for c in range(_NCH):
            sl = slice(c * 128, (c + 1) * 128)
            xv = ct_ref[0, :, sl]
            yv = ct_ref[1, :, sl]
            zv = ct_ref[2, :, sl]
            dx = xv - lx
            dy = yv - ly
            dz = zv - lz
            d = dx * dx + dy * dy + dz * dz
            dm = jnp.minimum(d_ref[:, sl], d)
            d_ref[:, sl] = dm
            better = dm > macc
            macc = jnp.where(better, dm, macc)
            bx = jnp.where(better, xv, bx)
            by = jnp.where(better, yv, by)
            bz = jnp.where(better, zv, bz)
            bc = jnp.where(better, jnp.int32(c), bc)
        m = jnp.max(macc, axis=1, keepdims=True)
        key = jnp.where(macc == m, bc * 128 + lane128, jnp.int32(_BIGI))
        kmin = jnp.min(key, axis=1, keepdims=True)
        sel2 = key == kmin
        nlx = jnp.sum(jnp.where(sel2, bx, 0.0), axis=1, keepdims=True)
        nly = jnp.sum(jnp.where(sel2, by, 0.0), axis=1, keepdims=True)
        nlz = jnp.sum(jnp.where(sel2, bz, 0.0), axis=1, keepdims=True)
        put = lane128 == j
        ax = jnp.where(put, nlx, ax)
        ay = jnp.where(put, nly, ay)
        az = jnp.where(put, nlz, az)
        return (nlx, nly, nlz, ax, ay, az)

    for ch in range(S // _FCH):
        start = 1 if ch == 0 else 0
        carry = (lx, ly, lz, ax, ay, az)
        carry = lax.fori_loop(start, _FCH, body, carry)
        lx, ly, lz, ax, ay, az = carry
        cs = ch * _FCH
        sel_ref[:, 0, cs:cs + _FCH] = ax
        sel_ref[:, 1, cs:cs + _FCH] = ay
        sel_ref[:, 2, cs:cs + _FCH] = az
        ax = zero_ch
        ay = zero_ch
        az = zero_ch


def _fps_z(ct, x, W):
    return pl.pallas_call(
        _fps_body,
        out_shape=[
            jax.ShapeDtypeStruct((B, 3, S), jnp.float32),
            jax.ShapeDtypeStruct((B, N, C), jnp.float32),
        ],
        scratch_shapes=[pltpu.VMEM((B, N), jnp.float32)],
    )(ct, x, W)


_SQ = 128


def _knn_body(q_ref, k_ref, idx_ref, d2_s):
    b = pl.program_id(0)
    q = q_ref[0]
    keys = k_ref[0]
    qT = jnp.transpose(q, (1, 0))
    g = lax.dot_general(qT, keys, (((1,), (0,)), ((), ())),
                        preferred_element_type=jnp.float32)
    qq = jnp.sum(qT * qT, axis=1, keepdims=True)
    pp = jnp.sum(keys * keys, axis=0, keepdims=True)
    d2_s[...] = qq + pp - 2.0 * g
    base = b * N
    inf = jnp.float32(jnp.inf)

    A1 = jnp.full((_SQ, 128), inf, dtype=jnp.float32)
    A2 = A1
    A3 = A1
    A4 = A1
    I1 = jnp.zeros((_SQ, 128), dtype=jnp.int32)
    I2 = I1
    I3 = I1
    I4 = I1
    for c in range(_NCH):
        v = d2_s[:, c * 128:(c + 1) * 128]
        cc = jnp.int32(c)
        c1 = v < A1
        c2 = v < A2
        c3 = v < A3
        c4 = v < A4
        A4 = jnp.where(c4, jnp.where(c3, A3, v), A4)
        I4 = jnp.where(c4, jnp.where(c3, I3, cc), I4)
        A3 = jnp.where(c3, jnp.where(c2, A2, v), A3)
        I3 = jnp.where(c3, jnp.where(c2, I2, cc), I3)
        A2 = jnp.where(c2, jnp.where(c1, A1, v), A2)
        I2 = jnp.where(c2, jnp.where(c1, I1, cc), I2)
        A1 = jnp.where(c1, v, A1)
        I1 = jnp.where(c1, cc, I1)

    lane128 = lax.broadcasted_iota(jnp.int32, (_SQ, 128), 1)
    K1 = I1 * 128 + lane128
    K2 = I2 * 128 + lane128
    K3 = I3 * 128 + lane128
    K4 = I4 * 128 + lane128
    m16 = None
    k16 = None
    for j in range(K):
        v = A1
        k = K1
        for Ai, Ki in ((A2, K2), (A3, K3), (A4, K4)):
            bet = (Ai < v) | ((Ai == v) & (Ki < k))
            v = jnp.where(bet, Ai, v)
            k = jnp.where(bet, Ki, k)
        m = jnp.min(v, axis=1, keepdims=True)
        kk = jnp.where(v == m, k, jnp.int32(_BIGI))
        kmin = jnp.min(kk, axis=1, keepdims=True)
        idx_ref[0, :, j:j + 1] = kmin + base
        h1 = (K1 == kmin) & (A1 == m)
        h2 = (K2 == kmin) & (A2 == m)
        h3 = (K3 == kmin) & (A3 == m)
        h4 = (K4 == kmin) & (A4 == m)
        A1 = jnp.where(h1, inf, A1)
        A2 = jnp.where(h2, inf, A2)
        A3 = jnp.where(h3, inf, A3)
        A4 = jnp.where(h4, inf, A4)
        m16 = m
        k16 = kmin

    lane = lax.broadcasted_iota(jnp.int32, (_SQ, N), 1)
    d2 = d2_s[...]
    less = (d2 < m16) | ((d2 == m16) & (lane < k16))
    cnt = jnp.sum(jnp.where(less, jnp.int32(1), jnp.int32(0)),
                  axis=1, keepdims=True)
    bad = jnp.max(jnp.abs(cnt - 15)) > 0

    @pl.when(bad)
    def _slow():
        for j in range(K):
            d2 = d2_s[...]
            m = jnp.min(d2, axis=1, keepdims=True)
            cand = jnp.where(d2 == m, lane, N)
            nxt = jnp.min(cand, axis=1, keepdims=True)
            idx_ref[0, :, j:j + 1] = nxt + base
            d2_s[...] = jnp.where(lane == nxt, inf, d2)


def _knn(fps_bcs, coords):
    return pl.pallas_call(
        _knn_body,
        grid=(B, S // _SQ),
        in_specs=[
            pl.BlockSpec((1, 3, _SQ), lambda b, s: (b, 0, s)),
            pl.BlockSpec((1, 3, N), lambda b, s: (b, 0, 0)),
        ],
        out_specs=pl.BlockSpec((1, _SQ, K), lambda b, s: (b, s, 0)),
        out_shape=jax.ShapeDtypeStruct((B, S, K), jnp.int32),
        scratch_shapes=[pltpu.VMEM((_SQ, N), jnp.float32)],
    )(fps_bcs, coords)


_NW = 32
_RPW = (B * S) // _NW
_GRP = 8
_NG = _RPW // _GRP


def _gmax_body(z_hbm, idx_hbm, y_hbm, idx_v, rows_v, out_v, sem):
    cid = lax.axis_index("c")
    sid = lax.axis_index("s")
    wid = sid * 2 + cid
    base = wid * _RPW
    pltpu.sync_copy(idx_hbm.at[pl.ds(base * K, _RPW * K)], idx_v)

    def group(g, _):
        ib = pl.multiple_of(g * (_GRP * K), 8)
        cp = pltpu.async_copy(
            z_hbm.at[idx_v.at[pl.ds(ib, _GRP * K)]], rows_v, sem)
        cp.wait()
        ob = g * _GRP
        for r in range(_GRP):
            for c4 in range(C // 16):
                cs = c4 * 16
                acc = rows_v[r * K, pl.ds(cs, 16)]
                for t in range(1, K):
                    acc = jnp.maximum(acc, rows_v[r * K + t, pl.ds(cs, 16)])
                out_v[ob + r, pl.ds(cs, 16)] = acc
        return 0

    lax.fori_loop(0, _NG, group, 0)
    pltpu.sync_copy(out_v, y_hbm.at[pl.ds(base, _RPW)])


@functools.cache
def _gmax_kernel():
    return pl.kernel(
        _gmax_body,
        out_type=jax.ShapeDtypeStruct((B * S, C), jnp.float32),
        mesh=plsc.VectorSubcoreMesh(
            core_axis_name="c", subcore_axis_name="s"),
        scratch_types=[
            pltpu.VMEM((_RPW * K,), jnp.int32),
            pltpu.VMEM((_GRP * K, C), jnp.float32),
            pltpu.VMEM((_RPW, C), jnp.float32),
            pltpu.SemaphoreType.DMA,
        ],
        compiler_params=pltpu.CompilerParams(use_tc_tiling_on_sc=False),
    )


@jax.jit
def kernel(x, coords, W):
    ct = jnp.transpose(coords, (1, 0, 2))
    fps_coords, z = _fps_z(ct, x, W)
    knn = _knn(fps_coords, coords)
    y_rows = _gmax_kernel()(z.reshape(B * N, C), knn.reshape(B * S * K))
    y = jnp.transpose(y_rows.reshape(B, S, C), (0, 2, 1))
    return (y, fps_coords)


# speedup vs baseline: 32.8756x; 1.0015x over previous
import functools

import jax
import jax.numpy as jnp
from jax import lax
from jax.experimental import pallas as pl
from jax.experimental.pallas import tpu as pltpu
from jax.experimental.pallas import tpu_sc as plsc

B = 8
C = 64
N = 8192
S = 1024
K = 16

_NCH = N // 128
_FCH = 128
_BIGI = 0x7FFFFFFF


def _fps_body(ct_ref, x_ref, w_ref, sel_ref, z_ref, d_ref):
    for b in range(B):
        z_ref[b] = lax.dot_general(
            x_ref[b], w_ref[...], (((0,), (1,)), ((), ())),
            preferred_element_type=jnp.float32)

    lane128 = lax.broadcasted_iota(jnp.int32, (B, 128), 1)
    d_ref[...] = jnp.full((B, N), 1e10, dtype=jnp.float32)

    lx = ct_ref[0, :, 0:1]
    ly = ct_ref[1, :, 0:1]
    lz = ct_ref[2, :, 0:1]
    zero_ch = jnp.zeros((B, _FCH), dtype=jnp.float32)
    put0 = lane128 == 0
    ax = jnp.where(put0, lx, zero_ch)
    ay = jnp.where(put0, ly, zero_ch)
    az = jnp.where(put0, lz, zero_ch)

    def body(j, carry):
        lx, ly, lz, ax, ay, az = carry
        macc = jnp.full((B, 128), -jnp.inf, dtype=jnp.float32)
        bx = zero_ch
        by = zero_ch
        bz = zero_ch
        bc = jnp.zeros((B, 128), dtype=jnp.int32)
        for c in range(_NCH):
            sl = slice(c * 128, (c + 1) * 128)
            xv = ct_ref[0, :, sl]
            yv = ct_ref[1, :, sl]
            zv = ct_ref[2, :, sl]
            dx = xv - lx
            dy = yv - ly
            dz = zv - lz
            d = dx * dx + dy * dy + dz * dz
            dm = jnp.minimum(d_ref[:, sl], d)
            d_ref[:, sl] = dm
            better = dm > macc
            macc = jnp.where(better, dm, macc)
            bx = jnp.where(better, xv, bx)
            by = jnp.where(better, yv, by)
            bz = jnp.where(better, zv, bz)
            bc = jnp.where(better, jnp.int32(c), bc)
        m = jnp.max(macc, axis=1, keepdims=True)
        key = jnp.where(macc == m, bc * 128 + lane128, jnp.int32(_BIGI))
        kmin = jnp.min(key, axis=1, keepdims=True)
        sel2 = key == kmin
        nlx = jnp.sum(jnp.where(sel2, bx, 0.0), axis=1, keepdims=True)
        nly = jnp.sum(jnp.where(sel2, by, 0.0), axis=1, keepdims=True)
        nlz = jnp.sum(jnp.where(sel2, bz, 0.0), axis=1, keepdims=True)
        put = lane128 == j
        ax = jnp.where(put, nlx, ax)
        ay = jnp.where(put, nly, ay)
        az = jnp.where(put, nlz, az)
        return (nlx, nly, nlz, ax, ay, az)

    for ch in range(S // _FCH):
        start = 1 if ch == 0 else 0
        carry = (lx, ly, lz, ax, ay, az)
        carry = lax.fori_loop(start, _FCH, body, carry)
        lx, ly, lz, ax, ay, az = carry
        cs = ch * _FCH
        sel_ref[:, 0, cs:cs + _FCH] = ax
        sel_ref[:, 1, cs:cs + _FCH] = ay
        sel_ref[:, 2, cs:cs + _FCH] = az
        ax = zero_ch
        ay = zero_ch
        az = zero_ch


def _fps_z(ct, x, W):
    return pl.pallas_call(
        _fps_body,
        out_shape=[
            jax.ShapeDtypeStruct((B, 3, S), jnp.float32),
            jax.ShapeDtypeStruct((B, N, C), jnp.float32),
        ],
        scratch_shapes=[pltpu.VMEM((B, N), jnp.float32)],
    )(ct, x, W)


_SQ = 128


def _knn_body(q_ref, k_ref, idx_ref, d2_s):
    b = pl.program_id(0)
    q = q_ref[0]
    keys = k_ref[0]
    qT = jnp.transpose(q, (1, 0))
    g = lax.dot_general(qT, keys, (((1,), (0,)), ((), ())),
                        preferred_element_type=jnp.float32)
    qq = jnp.sum(qT * qT, axis=1, keepdims=True)
    pp = jnp.sum(keys * keys, axis=0, keepdims=True)
    d2_s[...] = qq + pp - 2.0 * g
    base = b * N
    inf = jnp.float32(jnp.inf)

    A1 = jnp.full((_SQ, 128), inf, dtype=jnp.float32)
    A2 = A1
    A3 = A1
    A4 = A1
    I1 = jnp.zeros((_SQ, 128), dtype=jnp.int32)
    I2 = I1
    I3 = I1
    I4 = I1
    for c in range(_NCH):
        v = d2_s[:, c * 128:(c + 1) * 128]
        cc = jnp.int32(c)
        c1 = v < A1
        c2 = v < A2
        c3 = v < A3
        c4 = v < A4
        A4 = jnp.where(c4, jnp.where(c3, A3, v), A4)
        I4 = jnp.where(c4, jnp.where(c3, I3, cc), I4)
        A3 = jnp.where(c3, jnp.where(c2, A2, v), A3)
        I3 = jnp.where(c3, jnp.where(c2, I2, cc), I3)
        A2 = jnp.where(c2, jnp.where(c1, A1, v), A2)
        I2 = jnp.where(c2, jnp.where(c1, I1, cc), I2)
        A1 = jnp.where(c1, v, A1)
        I1 = jnp.where(c1, cc, I1)

    lane128 = lax.broadcasted_iota(jnp.int32, (_SQ, 128), 1)
    K1 = I1 * 128 + lane128
    K2 = I2 * 128 + lane128
    K3 = I3 * 128 + lane128
    K4 = I4 * 128 + lane128
    m16 = None
    k16 = None
    for j in range(K):
        v = A1
        k = K1
        for Ai, Ki in ((A2, K2), (A3, K3), (A4, K4)):
            bet = (Ai < v) | ((Ai == v) & (Ki < k))
            v = jnp.where(bet, Ai, v)
            k = jnp.where(bet, Ki, k)
        m = jnp.min(v, axis=1, keepdims=True)
        kk = jnp.where(v == m, k, jnp.int32(_BIGI))
        kmin = jnp.min(kk, axis=1, keepdims=True)
        idx_ref[0, :, j:j + 1] = kmin + base
        h1 = (K1 == kmin) & (A1 == m)
        h2 = (K2 == kmin) & (A2 == m)
        h3 = (K3 == kmin) & (A3 == m)
        h4 = (K4 == kmin) & (A4 == m)
        A1 = jnp.where(h1, inf, A1)
        A2 = jnp.where(h2, inf, A2)
        A3 = jnp.where(h3, inf, A3)
        A4 = jnp.where(h4, inf, A4)
        m16 = m
        k16 = kmin

    lane = lax.broadcasted_iota(jnp.int32, (_SQ, N), 1)
    d2 = d2_s[...]
    less = (d2 < m16) | ((d2 == m16) & (lane < k16))
    cnt = jnp.sum(jnp.where(less, jnp.int32(1), jnp.int32(0)),
                  axis=1, keepdims=True)
    bad = jnp.max(jnp.abs(cnt - 15)) > 0

    @pl.when(bad)
    def _slow():
        for j in range(K):
            d2 = d2_s[...]
            m = jnp.min(d2, axis=1, keepdims=True)
            cand = jnp.where(d2 == m, lane, N)
            nxt = jnp.min(cand, axis=1, keepdims=True)
            idx_ref[0, :, j:j + 1] = nxt + base
            d2_s[...] = jnp.where(lane == nxt, inf, d2)


def _knn(fps_bcs, coords):
    return pl.pallas_call(
        _knn_body,
        grid=(B, S // _SQ),
        in_specs=[
            pl.BlockSpec((1, 3, _SQ), lambda b, s: (b, 0, s)),
            pl.BlockSpec((1, 3, N), lambda b, s: (b, 0, 0)),
        ],
        out_specs=pl.BlockSpec((1, _SQ, K), lambda b, s: (b, s, 0)),
        out_shape=jax.ShapeDtypeStruct((B, S, K), jnp.int32),
        scratch_shapes=[pltpu.VMEM((_SQ, N), jnp.float32)],
    )(fps_bcs, coords)


_NW = 32
_RPW = (B * S) // _NW
_GRP = 8
_NG = _RPW // _GRP


def _gmax_body(z_hbm, idx_hbm, y_hbm, idx_v, rows_v, out_v, sem):
    cid = lax.axis_index("c")
    sid = lax.axis_index("s")
    wid = sid * 2 + cid
    base = wid * _RPW
    pltpu.sync_copy(idx_hbm.at[pl.ds(base * K, _RPW * K)], idx_v)

    def group(g, _):
        ib = pl.multiple_of(g * (_GRP * K), 8)
        cp = pltpu.async_copy(
            z_hbm.at[idx_v.at[pl.ds(ib, _GRP * K)]], rows_v, sem)
        cp.wait()
        ob = g * _GRP
        for r in range(_GRP):
            for c4 in range(C // 16):
                cs = c4 * 16
                acc = rows_v[r * K, pl.ds(cs, 16)]
                for t in range(1, K):
                    acc = jnp.maximum(acc, rows_v[r * K + t, pl.ds(cs, 16)])
                out_v[ob + r, pl.ds(cs, 16)] = acc
        return 0

    lax.fori_loop(0, _NG, group, 0)
    pltpu.sync_copy(out_v, y_hbm.at[pl.ds(base, _RPW)])


@functools.cache
def _gmax_kernel():
    return pl.kernel(
        _gmax_body,
        out_type=jax.ShapeDtypeStruct((B * S, C), jnp.float32),
        mesh=plsc.VectorSubcoreMesh(
            core_axis_name="c", subcore_axis_name="s"),
        scratch_types=[
            pltpu.VMEM((_RPW * K,), jnp.int32),
            pltpu.VMEM((_GRP * K, C), jnp.float32),
            pltpu.VMEM((_RPW, C), jnp.float32),
            pltpu.SemaphoreType.DMA,
        ],
        compiler_params=pltpu.CompilerParams(use_tc_tiling_on_sc=False),
    )


@jax.jit
def kernel(x, coords, W):
    ct = jnp.transpose(coords, (1, 0, 2))
    fps_coords, z = _fps_z(ct, x, W)
    idx = _knn(fps_coords, coords)
    y_rows = _gmax_kernel()(z.reshape(B * N, C), idx.reshape(B * S * K))
    y = jnp.transpose(y_rows.reshape(B, S, C), (0, 2, 1))
    return (y, fps_coords)
